# baseline (speedup 1.0000x reference)
import jax
import jax.numpy as jnp
from jax import lax
from jax.experimental import pallas as pl
from jax.experimental.pallas import tpu as pltpu
from jax.experimental.pallas import tpu_sc as plsc

N = 10000
E = 320000
D = 128
DE = 16
H = 64
EPS = 1e-5

NC = 2
NS = 16
NW = NC * NS
B = 128
NROWS = E // B
RPW_MAX = NROWS // NW + 1
PAIRS = NROWS // NW // 2
N_PAD = 10240
RPT = N_PAD // NS

_f32 = jnp.float32
_bf16 = jnp.bfloat16


_SC_MESH = plsc.VectorSubcoreMesh(core_axis_name="c", subcore_axis_name="s")
_SC_PARAMS = pltpu.CompilerParams(use_tc_tiling_on_sc=False)


def _make_sc_layer(with_deg: bool):
    out_type = [jax.ShapeDtypeStruct((NC, N_PAD, H), _bf16)]
    if with_deg:
        out_type.append(jax.ShapeDtypeStruct((NC, N_PAD, 16), _bf16))
    scratch_types = [
        pltpu.VMEM((RPW_MAX, B), jnp.int32),
        pltpu.VMEM((RPW_MAX, B), jnp.int32),
        pltpu.VMEM((B, H), _bf16),
        pltpu.VMEM((B, H), _bf16),
        pltpu.VMEM((B // 8, 8 * H), _bf16),
        pltpu.VMEM((B // 8, 8 * H), _bf16),
        pltpu.VMEM_SHARED((N_PAD, H), _bf16),
    ]
    if with_deg:
        scratch_types.append(pltpu.VMEM((B, 16), _bf16))
        scratch_types.append(pltpu.VMEM_SHARED((N_PAD, 16), _bf16))
    scratch_types += [pltpu.SemaphoreType.DMA] * 4

    def body(table, eaw, srcm, dstm, *rest):
        if with_deg:
            (s_out, deg_out, idx_s, idx_d, rows0, rows1, ea0, ea1, s_sh,
             ones_v, deg_sh, sg0, sg1, se0, se1) = rest
        else:
            (s_out, idx_s, idx_d, rows0, rows1, ea0, ea1, s_sh,
             sg0, sg1, se0, se1) = rest
            deg_out = ones_v = deg_sh = None
        rows = (rows0, rows1)
        ea = (ea0, ea1)
        sg = (sg0, sg1)
        se = (se0, se1)

        cid = lax.axis_index("c")
        sid = lax.axis_index("s")
        wid = sid * NC + cid
        lo = wid * NROWS // NW
        nr = (wid + 1) * NROWS // NW - lo
        zero32 = jnp.zeros((32,), _bf16)

        pltpu.sync_copy(srcm.at[pl.ds(lo, RPW_MAX)], idx_s)
        pltpu.sync_copy(dstm.at[pl.ds(lo, RPW_MAX)], idx_d)

        def zrow(j, _):
            for k in range(H // 32):
                rows0[j, pl.ds(k * 32, 32)] = zero32
            return 0
        lax.fori_loop(0, B, zrow, 0, unroll=4)

        base_r = sid * RPT
        for r0 in range(0, RPT, B):
            pltpu.sync_copy(rows0, s_sh.at[pl.ds(base_r + r0, B)])
        if with_deg:
            z216 = jnp.zeros((2, 16), _bf16)

            def zo(j, _):
                ones_v[pl.ds(2 * j, 2), :] = z216
                return 0
            lax.fori_loop(0, B // 2, zo, 0, unroll=4)
            for t in range(RPT // B):
                pltpu.sync_copy(ones_v, deg_sh.at[pl.ds(base_r + t * B, B)])
            o216 = jnp.full((2, 16), 1.0, _bf16)

            def oo(j, _):
                ones_v[pl.ds(2 * j, 2), :] = o216
                return 0
            lax.fori_loop(0, B // 2, oo, 0, unroll=4)

        plsc.subcore_barrier()

        def issue(r, p):
            pltpu.async_copy(eaw.at[pl.ds((lo + r) * (B // 8), B // 8)],
                             ea[p], se[p])
            pltpu.async_copy(table.at[idx_s.at[r]], rows[p], sg[p])

        def consume(r, p):
            pltpu.make_async_copy(table.at[pl.ds(0, B)], rows[p], sg[p]).wait()
            pltpu.make_async_copy(eaw.at[pl.ds(0, B // 8)], ea[p], se[p]).wait()

            def compute(j, _):
                for q in range(16):
                    sl32 = pl.ds(q * 32, 32)
                    sl = pl.ds((q % 2) * 32, 32)
                    jj = j * 8 + q // 2
                    rows[p][jj, sl] = jnp.maximum(
                        rows[p][jj, sl] + ea[p][j, sl32], 0.0)
                return 0
            lax.fori_loop(0, B // 8, compute, 0, unroll=2)

            pltpu.sync_copy(rows[p], s_sh.at[idx_d.at[r]], add=True)
            if with_deg:
                pltpu.sync_copy(ones_v, deg_sh.at[idx_d.at[r]], add=True)

        issue(0, 0)

        def step(t, _):
            r = 2 * t
            issue(r + 1, 1)
            consume(r, 0)

            @pl.when(r + 2 < nr)
            def _():
                issue(r + 2, 0)
            consume(r + 1, 1)
            return 0
        lax.fori_loop(0, PAIRS, step, 0)

        @pl.when(nr % 2 == 1)
        def _():
            consume(nr - 1, 0)
        plsc.subcore_barrier()

        for r0 in range(0, RPT, B):
            pltpu.sync_copy(s_sh.at[pl.ds(base_r + r0, B)], rows0)
            pltpu.sync_copy(rows0, s_out.at[cid, pl.ds(base_r + r0, B)])
        if with_deg:
            for t in range(RPT // B):
                r = base_r + t * B
                pltpu.sync_copy(deg_sh.at[pl.ds(r, B)], ones_v)
                pltpu.sync_copy(ones_v, deg_out.at[cid, pl.ds(r, B)])

    return pl.kernel(body, out_type=out_type, mesh=_SC_MESH,
                     scratch_types=scratch_types,
                     compiler_params=_SC_PARAMS)


_sc_layer_deg = _make_sc_layer(True)
_sc_layer = _make_sc_layer(False)


def _dot(a, b):
    return jnp.dot(a, b, preferred_element_type=_f32,
                   precision=lax.Precision.HIGHEST)


def _kx_body(x_ref, w_ref, o1_ref, o2_ref):
    y = _dot(x_ref[...], w_ref[...])
    o1_ref[...] = y[:, :H].astype(_bf16)
    o2_ref[...] = y[:, H:]


def _kea_body(ea_ref, w1_ref, w2_ref, b1_ref, b2_ref, o1_ref, o2_ref):
    a = ea_ref[...]
    o1_ref[...] = (jnp.dot(a, w1_ref[...], preferred_element_type=_f32)
                   + b1_ref[...]).astype(_bf16)
    o2_ref[...] = (jnp.dot(a, w2_ref[...], preferred_element_type=_f32)
                   + b2_ref[...]).astype(_bf16)


def _kmid_body(s_ref, deg_ref, xr_ref, g_ref, b_ref, w_ref, o1_ref, o2_ref):
    s = s_ref[0, :N, :].astype(_f32) + s_ref[1, :N, :].astype(_f32)
    deg = deg_ref[0, :N, 0:1].astype(_f32) + deg_ref[1, :N, 0:1].astype(_f32)
    h = xr_ref[...] + s / jnp.maximum(deg, 1.0)
    mu = jnp.mean(h, axis=0, keepdims=True)
    var = jnp.mean((h - mu) ** 2, axis=0, keepdims=True)
    h = (h - mu) * lax.rsqrt(var + EPS) * g_ref[...] + b_ref[...]
    h = jnp.maximum(h, 0.0)
    y = _dot(h, w_ref[...])
    o1_ref[...] = y[:, :H].astype(_bf16)
    o2_ref[...] = y[:, H:]


def _kfin_body(s_ref, deg_ref, xr_ref, g_ref, b_ref, wo_ref, bo_ref, o_ref):
    s = s_ref[0, :N, :].astype(_f32) + s_ref[1, :N, :].astype(_f32)
    deg = deg_ref[0, :N, 0:1].astype(_f32) + deg_ref[1, :N, 0:1].astype(_f32)
    h = xr_ref[...] + s / jnp.maximum(deg, 1.0)
    mu = jnp.mean(h, axis=0, keepdims=True)
    var = jnp.mean((h - mu) ** 2, axis=0, keepdims=True)
    h = (h - mu) * lax.rsqrt(var + EPS) * g_ref[...] + b_ref[...]
    h = jnp.maximum(h, 0.0)
    hg = jnp.mean(h, axis=0, keepdims=True)
    o_ref[...] = _dot(hg, wo_ref[...]) + bo_ref[...]


E8 = E // 8
_EA_BLK = 4000


def _run_tc(x, Wcat1, ea8, W81, W82, b81, b82):
    xW1, xroot1 = pl.pallas_call(
        _kx_body,
        out_shape=[jax.ShapeDtypeStruct((N, H), _bf16),
                   jax.ShapeDtypeStruct((N, H), _f32)],
    )(x, Wcat1)
    eaW1, eaW2 = pl.pallas_call(
        _kea_body,
        grid=(E8 // _EA_BLK,),
        in_specs=[pl.BlockSpec((_EA_BLK, D), lambda i: (i, 0)),
                  pl.BlockSpec((D, 8 * H), lambda i: (0, 0)),
                  pl.BlockSpec((D, 8 * H), lambda i: (0, 0)),
                  pl.BlockSpec((1, 8 * H), lambda i: (0, 0)),
                  pl.BlockSpec((1, 8 * H), lambda i: (0, 0))],
        out_specs=[pl.BlockSpec((_EA_BLK, 8 * H), lambda i: (i, 0)),
                   pl.BlockSpec((_EA_BLK, 8 * H), lambda i: (i, 0))],
        out_shape=[jax.ShapeDtypeStruct((E8, 8 * H), _bf16),
                   jax.ShapeDtypeStruct((E8, 8 * H), _bf16)],
    )(ea8, W81, W82, b81, b82)
    return xW1, xroot1, eaW1, eaW2


def kernel(x, edge_index, edge_attr, W_neigh1, b_neigh1, W_root1, gamma1,
           beta1, W_neigh2, b_neigh2, W_root2, gamma2, beta2, W_out, b_out):
    srcm = edge_index[0].astype(jnp.int32).reshape(NROWS, B)
    dstm = edge_index[1].astype(jnp.int32).reshape(NROWS, B)

    Wcat1 = jnp.concatenate([W_neigh1[:D], W_root1], axis=1)
    Wcat2 = jnp.concatenate([W_neigh2[:H], W_root2], axis=1)
    eye8 = jnp.eye(8, dtype=_f32)
    W81 = jnp.kron(eye8, W_neigh1[D:])
    W82 = jnp.kron(eye8, W_neigh2[H:])
    b81 = jnp.tile(b_neigh1, 8).reshape(1, 8 * H)
    b82 = jnp.tile(b_neigh2, 8).reshape(1, 8 * H)
    ea8 = edge_attr.reshape(E // 8, 8 * DE)

    xW1, xroot1, eaW1, eaW2 = _run_tc(x, Wcat1, ea8, W81, W82, b81, b82)

    s1p, degp = _sc_layer_deg(xW1, eaW1, srcm, dstm)

    xW2, hroot2 = pl.pallas_call(
        _kmid_body,
        out_shape=[jax.ShapeDtypeStruct((N, H), _bf16),
                   jax.ShapeDtypeStruct((N, H), _f32)],
    )(s1p, degp, xroot1, gamma1.reshape(1, H), beta1.reshape(1, H), Wcat2)

    (s2p,) = _sc_layer(xW2, eaW2, srcm, dstm)

    out = pl.pallas_call(
        _kfin_body,
        out_shape=jax.ShapeDtypeStruct((1, 1), _f32),
    )(s2p, degp, hroot2, gamma2.reshape(1, H), beta2.reshape(1, H),
      W_out, b_out.reshape(1, 1))
    return out.reshape(1)
